# Initial kernel scaffold; baseline (speedup 1.0000x reference)
import functools

import jax
import jax.numpy as jnp
from jax import lax
from jax.experimental import pallas as pl
from jax.experimental.pallas import tpu as pltpu
from jax.experimental.pallas import tpu_sc as plsc

D_MODEL = 768
MAXLEN = 8192
SEQ = 2 * MAXLEN - 1
NW = 32
B_PAD = SEQ + 1
B_PER_W = B_PAD // NW
C = 64
NCH = B_PER_W // C
L = 16

_mesh = plsc.VectorSubcoreMesh(core_axis_name="c", subcore_axis_name="s")


@functools.partial(
    pl.kernel,
    out_type=jax.ShapeDtypeStruct((SEQ, D_MODEL), jnp.float32),
    mesh=_mesh,
    scratch_types=[
        pltpu.VMEM((B_PER_W,), jnp.int32),
        pltpu.VMEM((C, D_MODEL), jnp.float32),
        pltpu.VMEM((C, D_MODEL), jnp.float32),
        pltpu.SemaphoreType.DMA,
    ],
)
def _pe_gather(idx_hbm, table_hbm, out_hbm, idx_v, buf0, buf1, gsem):
    wid = lax.axis_index("s") * 2 + lax.axis_index("c")
    base = pl.multiple_of(wid * B_PER_W, B_PER_W)
    pltpu.sync_copy(idx_hbm.at[pl.ds(base, B_PER_W)], idx_v)
    for i in range(B_PER_W // L):
        p = idx_v[pl.ds(i * L, L)]
        p = jnp.minimum(jnp.maximum(p, -MAXLEN), MAXLEN - 1) + MAXLEN
        idx_v[pl.ds(i * L, L)] = p
    bufs = (buf0, buf1)
    for c in range(NCH):
        buf = bufs[c % 2]
        pltpu.async_copy(table_hbm.at[idx_v.at[pl.ds(c * C, C)]], buf, gsem).wait()
        if c < NCH - 1:
            pltpu.sync_copy(buf, out_hbm.at[pl.ds(base + c * C, C)])
        else:
            @pl.when(wid < NW - 1)
            def _full_store():
                pltpu.sync_copy(buf, out_hbm.at[pl.ds(base + c * C, C)])

            @pl.when(wid == NW - 1)
            def _tail_store():
                pltpu.sync_copy(
                    buf.at[pl.ds(0, C - 1)],
                    out_hbm.at[pl.ds(base + c * C, C - 1)],
                )


def kernel(pos_seq, W_k):
    idx = jnp.pad(pos_seq, (0, 1))
    return _pe_gather(idx, W_k)


# speedup vs baseline: 2.0374x; 2.0374x over previous
import functools

import jax
import jax.numpy as jnp
from jax import lax
from jax.experimental import pallas as pl
from jax.experimental.pallas import tpu as pltpu
from jax.experimental.pallas import tpu_sc as plsc

D_MODEL = 768
MAXLEN = 8192
SEQ = 2 * MAXLEN - 1
NW = 32
B_PAD = SEQ + 1
B_PER_W = B_PAD // NW
C = 64
NCH = B_PER_W // C
L = 16

_mesh = plsc.VectorSubcoreMesh(core_axis_name="c", subcore_axis_name="s")


@functools.partial(
    pl.kernel,
    out_type=jax.ShapeDtypeStruct((SEQ, D_MODEL), jnp.float32),
    mesh=_mesh,
    scratch_types=[
        pltpu.VMEM((B_PER_W,), jnp.int32),
        pltpu.VMEM((C, D_MODEL), jnp.float32),
        pltpu.VMEM((C, D_MODEL), jnp.float32),
        pltpu.VMEM((C,), jnp.int32),
        pltpu.SemaphoreType.DMA,
        pltpu.SemaphoreType.DMA,
    ],
)
def _pe_gather(idx_hbm, table_hbm, out_hbm, idx_v, buf0, buf1, scat_idx, gsem, ssem):
    wid = lax.axis_index("s") * 2 + lax.axis_index("c")
    base = pl.multiple_of(wid * B_PER_W, B_PER_W)
    pltpu.sync_copy(idx_hbm.at[pl.ds(base, B_PER_W)], idx_v)
    for i in range(B_PER_W // L):
        p = idx_v[pl.ds(i * L, L)]
        p = jnp.minimum(jnp.maximum(p, -MAXLEN), MAXLEN - 1) + MAXLEN
        idx_v[pl.ds(i * L, L)] = p
    bufs = (buf0, buf1)
    for c in range(NCH):
        buf = bufs[c % 2]
        pltpu.async_copy(table_hbm.at[idx_v.at[pl.ds(c * C, C)]], buf, gsem).wait()
        if c < NCH - 1:
            pltpu.sync_copy(buf, out_hbm.at[pl.ds(base + c * C, C)])
        else:
            @pl.when(wid < NW - 1)
            def _full_store():
                pltpu.sync_copy(buf, out_hbm.at[pl.ds(base + c * C, C)])

            @pl.when(wid == NW - 1)
            def _tail_store():
                for i in range(C // L):
                    v = lax.iota(jnp.int32, L) + (base + c * C + i * L)
                    scat_idx[pl.ds(i * L, L)] = jnp.minimum(v, SEQ - 1)
                pltpu.async_copy(buf, out_hbm.at[scat_idx], ssem).wait()


def kernel(pos_seq, W_k):
    idx = jnp.concatenate([pos_seq, pos_seq[-1:]])
    return _pe_gather(idx, W_k)


# speedup vs baseline: 2.2203x; 1.0898x over previous
import functools

import jax
import jax.numpy as jnp
from jax import lax
from jax.experimental import pallas as pl
from jax.experimental.pallas import tpu as pltpu
from jax.experimental.pallas import tpu_sc as plsc

D_MODEL = 768
MAXLEN = 8192
SEQ = 2 * MAXLEN - 1
NW = 32
B_PAD = SEQ + 1
B_PER_W = B_PAD // NW
C = 64
NCH = B_PER_W // C
L = 16

_mesh = plsc.VectorSubcoreMesh(core_axis_name="c", subcore_axis_name="s")


@functools.partial(
    pl.kernel,
    out_type=jax.ShapeDtypeStruct((SEQ, D_MODEL), jnp.float32),
    mesh=_mesh,
    scratch_types=[
        pltpu.VMEM((B_PER_W,), jnp.int32),
        pltpu.VMEM((C, D_MODEL), jnp.float32),
        pltpu.VMEM((C, D_MODEL), jnp.float32),
        pltpu.VMEM((C,), jnp.int32),
        pltpu.SemaphoreType.DMA,
        pltpu.SemaphoreType.DMA,
        pltpu.SemaphoreType.DMA,
        pltpu.SemaphoreType.DMA,
        pltpu.SemaphoreType.DMA,
    ],
)
def _pe_gather(idx_hbm, table_hbm, out_hbm, idx_v, buf0, buf1, scat_idx,
               gsem0, gsem1, ssem0, ssem1, tsem):
    wid = lax.axis_index("s") * 2 + lax.axis_index("c")
    base = pl.multiple_of(wid * B_PER_W, B_PER_W)
    pltpu.sync_copy(idx_hbm.at[pl.ds(base, B_PER_W)], idx_v)
    for i in range(B_PER_W // L):
        p = idx_v[pl.ds(i * L, L)]
        p = jnp.minimum(jnp.maximum(p, -MAXLEN), MAXLEN - 1) + MAXLEN
        idx_v[pl.ds(i * L, L)] = p
    bufs = (buf0, buf1)
    gsems = (gsem0, gsem1)
    ssems = (ssem0, ssem1)
    gathers = [None, None]
    stores = [None, None]

    def start_gather(c):
        b = c % 2
        gathers[b] = pltpu.async_copy(
            table_hbm.at[idx_v.at[pl.ds(c * C, C)]], bufs[b], gsems[b]
        )

    start_gather(0)
    for c in range(NCH):
        b = c % 2
        nb = (c + 1) % 2
        if c + 1 < NCH:
            if stores[nb] is not None:
                stores[nb].wait()
                stores[nb] = None
            start_gather(c + 1)
        gathers[b].wait()
        if c < NCH - 1:
            stores[b] = pltpu.async_copy(
                bufs[b], out_hbm.at[pl.ds(base + c * C, C)], ssems[b]
            )
        else:
            @pl.when(wid < NW - 1)
            def _full_store():
                pltpu.sync_copy(bufs[b], out_hbm.at[pl.ds(base + c * C, C)])

            @pl.when(wid == NW - 1)
            def _tail_store():
                for i in range(C // L):
                    v = lax.iota(jnp.int32, L) + (base + c * C + i * L)
                    scat_idx[pl.ds(i * L, L)] = jnp.minimum(v, SEQ - 1)
                pltpu.async_copy(bufs[b], out_hbm.at[scat_idx], tsem).wait()
    for h in stores:
        if h is not None:
            h.wait()


def kernel(pos_seq, W_k):
    idx = jnp.concatenate([pos_seq, pos_seq[-1:]])
    return _pe_gather(idx, W_k)
